# baseline (speedup 1.0000x reference)
import jax
import jax.numpy as jnp
from jax import lax
from jax.experimental import pallas as pl
from jax.experimental.pallas import tpu as pltpu
from jax.experimental.pallas import tpu_sc as plsc

_L = 16
_H_TILES = ((0, 1, 2), (3, 4, 5), (6, 7, 8), (9, 10, 11), (12, 13), (14, 15))


def _splat_i(val):
    return jnp.full((_L,), val, jnp.int32)


def _lane_bcast(vec, lane):
    dn = lax.GatherDimensionNumbers(
        offset_dims=(), collapsed_slice_dims=(0,), start_index_map=(0,)
    )
    return lax.gather(
        vec, _splat_i(lane)[:, None], dn, slice_sizes=(1,),
        mode=lax.GatherScatterMode.PROMISE_IN_BOUNDS,
    )


def _compute_row(xb, kb, ob, wv, bv, slot, ns, nj):
    ngrp = nj // _L
    brow = bv[...]
    for tile in _H_TILES:
        wsp = [[_lane_bcast(wv[h, :], s) for s in range(ns)] for h in tile]
        bsp = [_lane_bcast(brow, h) for h in tile]

        def grp_body(g, _, tile=tile, wsp=wsp, bsp=bsp):
            off = pl.multiple_of(g * _L, _L)
            kv = kb[slot, pl.ds(off, _L)]
            accs = list(bsp)
            for s in range(ns):
                xv = xb[slot, s, pl.ds(off, _L)]
                for t in range(len(tile)):
                    accs[t] = accs[t] + wsp[t][s] * xv
            for t, h in enumerate(tile):
                ob[slot, h, pl.ds(off, _L)] = accs[t] * kv
            return _

        lax.fori_loop(0, ngrp, grp_body, None)


def kernel(stacks, mask, W, b):
    bsz, ns, n, nj = stacks.shape
    nh = W.shape[0]
    keep = 1.0 - mask.astype(jnp.float32)

    mesh = plsc.VectorSubcoreMesh(core_axis_name="c", subcore_axis_name="s")
    info = plsc.get_sparse_core_info()
    nw = info.num_cores * info.num_subcores
    rows_w = (bsz * n) // nw

    def run(stacks, keep, W, b):
        def body(stacks_h, keep_h, w_h, b_h, out_h,
                 wv, bv, xb, kb, ob,
                 in_s0, in_s1, out_s0, out_s1):
            cid = lax.axis_index("c")
            sid = lax.axis_index("s")
            wid = sid * info.num_cores + cid
            base = wid * rows_w
            b_idx = base // n
            i0 = base % n
            in_sems = (in_s0, in_s1)
            out_sems = (out_s0, out_s1)

            pltpu.sync_copy(w_h, wv)
            pltpu.sync_copy(b_h, bv)

            def start_in(g, slot):
                i = i0 + g
                for s in range(ns):
                    pltpu.make_async_copy(
                        stacks_h.at[b_idx, s, i], xb.at[slot, s], in_sems[slot]
                    ).start()
                pltpu.make_async_copy(
                    keep_h.at[b_idx, i], kb.at[slot], in_sems[slot]
                ).start()

            def wait_in(g, slot):
                i = i0 + g
                for s in range(ns):
                    pltpu.make_async_copy(
                        stacks_h.at[b_idx, s, i], xb.at[slot, s], in_sems[slot]
                    ).wait()
                pltpu.make_async_copy(
                    keep_h.at[b_idx, i], kb.at[slot], in_sems[slot]
                ).wait()

            def start_out(g, slot):
                i = i0 + g
                for h in range(nh):
                    pltpu.make_async_copy(
                        ob.at[slot, h], out_h.at[b_idx, h, i], out_sems[slot]
                    ).start()

            def wait_out(g, slot):
                i = i0 + g
                for h in range(nh):
                    pltpu.make_async_copy(
                        ob.at[slot, h], out_h.at[b_idx, h, i], out_sems[slot]
                    ).wait()

            start_in(0, 0)

            def outer(g2, _):
                g = g2 * 2
                start_in(g + 1, 1)
                wait_in(g, 0)

                @pl.when(g2 > 0)
                def _():
                    wait_out(g - 2, 0)

                _compute_row(xb, kb, ob, wv, bv, 0, ns, nj)
                start_out(g, 0)

                @pl.when(g2 < rows_w // 2 - 1)
                def _():
                    start_in(g + 2, 0)

                wait_in(g + 1, 1)

                @pl.when(g2 > 0)
                def _():
                    wait_out(g - 1, 1)

                _compute_row(xb, kb, ob, wv, bv, 1, ns, nj)
                start_out(g + 1, 1)
                return _

            lax.fori_loop(0, rows_w // 2, outer, None)
            wait_out(rows_w - 2, 0)
            wait_out(rows_w - 1, 1)

        f = pl.kernel(
            body,
            out_type=jax.ShapeDtypeStruct((bsz, nh, n, nj), jnp.float32),
            mesh=mesh,
            scratch_types=[
                pltpu.VMEM((nh, ns), jnp.float32),
                pltpu.VMEM((nh,), jnp.float32),
                pltpu.VMEM((2, ns, nj), jnp.float32),
                pltpu.VMEM((2, nj), jnp.float32),
                pltpu.VMEM((2, nh, nj), jnp.float32),
                pltpu.SemaphoreType.DMA,
                pltpu.SemaphoreType.DMA,
                pltpu.SemaphoreType.DMA,
                pltpu.SemaphoreType.DMA,
            ],
        )
        return f(stacks, keep, W, b)

    return run(stacks, keep, W, b)


# speedup vs baseline: 3.5119x; 3.5119x over previous
import jax
import jax.numpy as jnp
from jax.experimental import pallas as pl
from jax.experimental.pallas import tpu as pltpu

_BR = 16


def _tc_body(x_ref, k_ref, w_ref, b_ref, o_ref):
    w = w_ref[...]
    bias = b_ref[...]
    for r in range(_BR):
        x = x_ref[0, :, r, :]
        y = jax.lax.dot_general(
            w, x, (((1,), (0,)), ((), ())),
            preferred_element_type=jnp.float32,
        )
        o_ref[0, :, r, :] = (y + bias) * k_ref[0, r][None, :]


def kernel(stacks, mask, W, b):
    bsz, num_stacks, n, n1 = stacks.shape
    nh = W.shape[0]
    keep = 1.0 - mask.astype(jnp.float32)
    b2 = b.reshape(nh, 1)
    grid = (bsz, n // _BR)
    out = pl.pallas_call(
        _tc_body,
        grid=grid,
        in_specs=[
            pl.BlockSpec((1, num_stacks, _BR, n1), lambda bi, ri: (bi, 0, ri, 0)),
            pl.BlockSpec((1, _BR, n1), lambda bi, ri: (bi, ri, 0)),
            pl.BlockSpec((nh, num_stacks), lambda bi, ri: (0, 0)),
            pl.BlockSpec((nh, 1), lambda bi, ri: (0, 0)),
        ],
        out_specs=pl.BlockSpec((1, nh, _BR, n1), lambda bi, ri: (bi, 0, ri, 0)),
        out_shape=jax.ShapeDtypeStruct((bsz, nh, n, n1), jnp.float32),
        compiler_params=pltpu.CompilerParams(
            dimension_semantics=("parallel", "parallel"),
        ),
    )(stacks, keep, W, b2)
    return out


# speedup vs baseline: 4.3171x; 1.2293x over previous
import jax
import jax.numpy as jnp
from jax.experimental import pallas as pl
from jax.experimental.pallas import tpu as pltpu

_BR = 32


def _tc_body(x_ref, k_ref, w_ref, b_ref, o_ref):
    w = w_ref[...]
    bias = b_ref[...]
    for r in range(_BR):
        x = x_ref[0, :, r, :]
        y = jax.lax.dot_general(
            w, x, (((1,), (0,)), ((), ())),
            preferred_element_type=jnp.float32,
        )
        o_ref[0, :, r, :] = (y + bias) * k_ref[0, r][None, :]


def kernel(stacks, mask, W, b):
    bsz, num_stacks, n, n1 = stacks.shape
    nh = W.shape[0]
    keep = 1.0 - mask.astype(jnp.float32)
    b2 = b.reshape(nh, 1)
    grid = (bsz, n // _BR)
    out = pl.pallas_call(
        _tc_body,
        grid=grid,
        in_specs=[
            pl.BlockSpec((1, num_stacks, _BR, n1), lambda bi, ri: (bi, 0, ri, 0)),
            pl.BlockSpec((1, _BR, n1), lambda bi, ri: (bi, ri, 0)),
            pl.BlockSpec((nh, num_stacks), lambda bi, ri: (0, 0)),
            pl.BlockSpec((nh, 1), lambda bi, ri: (0, 0)),
        ],
        out_specs=pl.BlockSpec((1, nh, _BR, n1), lambda bi, ri: (bi, 0, ri, 0)),
        out_shape=jax.ShapeDtypeStruct((bsz, nh, n, n1), jnp.float32),
        compiler_params=pltpu.CompilerParams(
            dimension_semantics=("parallel", "parallel"),
        ),
    )(stacks, keep, W, b2)
    return out


# speedup vs baseline: 4.8464x; 1.1226x over previous
import jax
import jax.numpy as jnp
from jax.experimental import pallas as pl
from jax.experimental.pallas import tpu as pltpu

_BR = 64


def _tc_body(x_ref, k_ref, w_ref, b_ref, o_ref):
    w = w_ref[...]
    bias = b_ref[...]
    for r in range(_BR):
        x = x_ref[0, :, r, :]
        y = jax.lax.dot_general(
            w, x, (((1,), (0,)), ((), ())),
            preferred_element_type=jnp.float32,
        )
        o_ref[0, :, r, :] = (y + bias) * k_ref[0, r][None, :]


def kernel(stacks, mask, W, b):
    bsz, num_stacks, n, n1 = stacks.shape
    nh = W.shape[0]
    keep = 1.0 - mask.astype(jnp.float32)
    b2 = b.reshape(nh, 1)
    grid = (bsz, n // _BR)
    out = pl.pallas_call(
        _tc_body,
        grid=grid,
        in_specs=[
            pl.BlockSpec((1, num_stacks, _BR, n1), lambda bi, ri: (bi, 0, ri, 0)),
            pl.BlockSpec((1, _BR, n1), lambda bi, ri: (bi, ri, 0)),
            pl.BlockSpec((nh, num_stacks), lambda bi, ri: (0, 0)),
            pl.BlockSpec((nh, 1), lambda bi, ri: (0, 0)),
        ],
        out_specs=pl.BlockSpec((1, nh, _BR, n1), lambda bi, ri: (bi, 0, ri, 0)),
        out_shape=jax.ShapeDtypeStruct((bsz, nh, n, n1), jnp.float32),
        compiler_params=pltpu.CompilerParams(
            dimension_semantics=("parallel", "parallel"),
        ),
    )(stacks, keep, W, b2)
    return out


# speedup vs baseline: 5.0921x; 1.0507x over previous
import jax
import jax.numpy as jnp
from jax.experimental import pallas as pl
from jax.experimental.pallas import tpu as pltpu

_BR = 128


def _tc_body(x_ref, k_ref, w_ref, b_ref, o_ref):
    w = w_ref[...]
    bias = b_ref[...]
    for r in range(_BR):
        x = x_ref[0, :, r, :]
        y = jax.lax.dot_general(
            w, x, (((1,), (0,)), ((), ())),
            preferred_element_type=jnp.float32,
        )
        o_ref[0, :, r, :] = (y + bias) * k_ref[0, r][None, :]


def kernel(stacks, mask, W, b):
    bsz, num_stacks, n, n1 = stacks.shape
    nh = W.shape[0]
    keep = 1.0 - mask.astype(jnp.float32)
    b2 = b.reshape(nh, 1)
    grid = (bsz, n // _BR)
    out = pl.pallas_call(
        _tc_body,
        grid=grid,
        in_specs=[
            pl.BlockSpec((1, num_stacks, _BR, n1), lambda bi, ri: (bi, 0, ri, 0)),
            pl.BlockSpec((1, _BR, n1), lambda bi, ri: (bi, ri, 0)),
            pl.BlockSpec((nh, num_stacks), lambda bi, ri: (0, 0)),
            pl.BlockSpec((nh, 1), lambda bi, ri: (0, 0)),
        ],
        out_specs=pl.BlockSpec((1, nh, _BR, n1), lambda bi, ri: (bi, 0, ri, 0)),
        out_shape=jax.ShapeDtypeStruct((bsz, nh, n, n1), jnp.float32),
        compiler_params=pltpu.CompilerParams(
            dimension_semantics=("parallel", "parallel"),
        ),
    )(stacks, keep, W, b2)
    return out
